# baseline (speedup 1.0000x reference)
import jax
import jax.numpy as jnp
from jax import lax
from jax.experimental import pallas as pl
from jax.experimental.pallas import tpu as pltpu
from jax.experimental.pallas import tpu_sc as plsc

_FRAME_NUMBER = 16
_FRAME_SIZE = 128
_SAMPLE_NUM = 8
_NUM_EVENTS = 1048576

_NBINS = _FRAME_SIZE * _FRAME_SIZE
_TW = _NUM_EVENTS // _FRAME_NUMBER
_CHUNK = 4096
_NCHUNK = _TW // _CHUNK
_ACC = _NBINS * 4


def _sc_body(ev_hbm, idx1_hbm, idx2_hbm, scale_hbm, out_hbm,
             bidx1, bidx2, bv0, bv1, bscale, acc, sem0, sem1):
    nc = 2
    wid = lax.axis_index("s") * nc + lax.axis_index("c")
    pair = wid % 4
    fbase = (wid // 4) * 2
    sems = (sem0, sem1)

    pltpu.sync_copy(scale_hbm, bscale)
    sc = bscale[...]
    zeros16 = jnp.zeros((16,), jnp.float32)

    def _start(f, cidx, buf):
        off = f * _TW + cidx * _CHUNK
        s = pl.ds(off, _CHUNK)
        sem = sems[buf]
        bslice = pl.ds(buf * _CHUNK, _CHUNK)
        return [
            pltpu.async_copy(idx1_hbm.at[s], bidx1.at[bslice], sem),
            pltpu.async_copy(idx2_hbm.at[s], bidx2.at[bslice], sem),
            pltpu.async_copy(ev_hbm.at[pl.ds(2 * pair, 1), s],
                             bv0.at[buf], sem),
            pltpu.async_copy(ev_hbm.at[pl.ds(2 * pair + 1, 1), s],
                             bv1.at[buf], sem),
        ]

    for fi in range(2):
        f = fbase + fi
        pending = _start(f, 0, 0)

        def _zero(i, _):
            acc[pl.ds(i * 16, 16)] = zeros16
            return ()

        lax.fori_loop(0, _ACC // 16, _zero, (), unroll=4)

        for cidx in range(_NCHUNK):
            buf = cidx % 2
            for h in pending:
                h.wait()
            if cidx + 1 < _NCHUNK:
                pending = _start(f, cidx + 1, 1 - buf)

            base = buf * _CHUNK
            w0, w1 = bv0.at[buf, 0], bv1.at[buf, 0]

            @plsc.parallel_loop(0, _CHUNK // 16, unroll=16)
            def _group(i):
                s = pl.ds(i * 16, 16)
                sb = pl.ds(base + i * 16, 16)
                b = bidx2[sb] * _FRAME_SIZE + bidx1[sb]
                v0 = w0[s] * sc
                v1 = w1[s] * sc
                p0 = jnp.where(v0 > 0.0, _NBINS, 0).astype(jnp.int32)
                p1 = jnp.where(v1 > 0.0, _NBINS, 0).astype(jnp.int32)
                plsc.addupdate_scatter(acc, [b + p0], jnp.abs(v0))
                plsc.addupdate_scatter(acc, [b + 2 * _NBINS + p1],
                                       jnp.abs(v1))

        for j in range(4):
            s_out = 2 * pair + j // 2
            dst = ((f * _SAMPLE_NUM + s_out) * 2 + j % 2) * _NBINS
            pltpu.sync_copy(acc.at[pl.ds(j * _NBINS, _NBINS)],
                            out_hbm.at[pl.ds(dst, _NBINS)])


@jax.jit
def _frame_hist(event_values, idx1, idx2, scale):
    mesh = plsc.VectorSubcoreMesh(core_axis_name="c", subcore_axis_name="s")
    return pl.kernel(
        _sc_body,
        out_type=jax.ShapeDtypeStruct(
            (_FRAME_NUMBER * _SAMPLE_NUM * 2 * _NBINS,), jnp.float32),
        mesh=mesh,
        compiler_params=pltpu.CompilerParams(needs_layout_passes=False),
        scratch_types=[
            pltpu.VMEM((2 * _CHUNK,), jnp.int32),
            pltpu.VMEM((2 * _CHUNK,), jnp.int32),
            pltpu.VMEM((2, 1, _CHUNK), jnp.float32),
            pltpu.VMEM((2, 1, _CHUNK), jnp.float32),
            pltpu.VMEM((16,), jnp.float32),
            pltpu.VMEM((_ACC,), jnp.float32),
            pltpu.SemaphoreType.DMA,
            pltpu.SemaphoreType.DMA,
        ],
    )(event_values, idx1, idx2, scale)


def kernel(event_values, event_indices, use_soft):
    scale = jnp.ones((16,), jnp.float32) * jnp.where(use_soft, 0.0, 1.0)
    idx1 = event_indices[0, 1]
    idx2 = event_indices[0, 2]
    raw = _frame_hist(event_values, idx1, idx2, scale)
    return raw.reshape(_FRAME_NUMBER, _SAMPLE_NUM, 2, _FRAME_SIZE, _FRAME_SIZE)


# speedup vs baseline: 1.0164x; 1.0164x over previous
import jax
import jax.numpy as jnp
from jax import lax
from jax.experimental import pallas as pl
from jax.experimental.pallas import tpu as pltpu
from jax.experimental.pallas import tpu_sc as plsc

_FRAME_NUMBER = 16
_FRAME_SIZE = 128
_SAMPLE_NUM = 8
_NUM_EVENTS = 1048576

_NBINS = _FRAME_SIZE * _FRAME_SIZE
_TW = _NUM_EVENTS // _FRAME_NUMBER
_CHUNK = 4096
_NCHUNK = _TW // _CHUNK
_ACC = _NBINS * 4


def _sc_body(ev_hbm, idx1_hbm, idx2_hbm, scale_hbm, out_hbm,
             bidx1, bidx2, bv0, bv1, bscale, acc, sem0, sem1):
    nc = 2
    wid = lax.axis_index("s") * nc + lax.axis_index("c")
    pair = wid % 4
    fbase = (wid // 4) * 2
    sems = (sem0, sem1)

    pltpu.sync_copy(scale_hbm, bscale)
    sc = bscale[...]
    zeros16 = jnp.zeros((16,), jnp.float32)

    def _start(f, cidx, buf):
        off = f * _TW + cidx * _CHUNK
        s = pl.ds(off, _CHUNK)
        sem = sems[buf]
        bslice = pl.ds(buf * _CHUNK, _CHUNK)
        return [
            pltpu.async_copy(idx1_hbm.at[s], bidx1.at[bslice], sem),
            pltpu.async_copy(idx2_hbm.at[s], bidx2.at[bslice], sem),
            pltpu.async_copy(ev_hbm.at[pl.ds(2 * pair, 1), s],
                             bv0.at[buf], sem),
            pltpu.async_copy(ev_hbm.at[pl.ds(2 * pair + 1, 1), s],
                             bv1.at[buf], sem),
        ]

    for fi in range(2):
        f = fbase + fi
        pending = _start(f, 0, 0)

        def _zero(i, _):
            acc[pl.ds(i * 16, 16)] = zeros16
            return ()

        lax.fori_loop(0, _ACC // 16, _zero, (), unroll=4)

        for cidx in range(_NCHUNK):
            buf = cidx % 2
            for h in pending:
                h.wait()
            if cidx + 1 < _NCHUNK:
                pending = _start(f, cidx + 1, 1 - buf)

            base = buf * _CHUNK
            w0, w1 = bv0.at[buf, 0], bv1.at[buf, 0]

            @plsc.parallel_loop(0, _CHUNK // 16, unroll=8)
            def _group(i):
                s = pl.ds(i * 16, 16)
                sb = pl.ds(base + i * 16, 16)
                b = bidx2[sb] * _FRAME_SIZE + bidx1[sb]
                v0 = w0[s] * sc
                v1 = w1[s] * sc
                p0 = jnp.where(v0 > 0.0, _NBINS, 0).astype(jnp.int32)
                p1 = jnp.where(v1 > 0.0, _NBINS, 0).astype(jnp.int32)
                plsc.addupdate_scatter(acc, [b + p0], jnp.abs(v0))
                plsc.addupdate_scatter(acc, [b + 2 * _NBINS + p1],
                                       jnp.abs(v1))

        for j in range(4):
            s_out = 2 * pair + j // 2
            dst = ((f * _SAMPLE_NUM + s_out) * 2 + j % 2) * _NBINS
            pltpu.sync_copy(acc.at[pl.ds(j * _NBINS, _NBINS)],
                            out_hbm.at[pl.ds(dst, _NBINS)])


@jax.jit
def _frame_hist(event_values, idx1, idx2, scale):
    mesh = plsc.VectorSubcoreMesh(core_axis_name="c", subcore_axis_name="s")
    return pl.kernel(
        _sc_body,
        out_type=jax.ShapeDtypeStruct(
            (_FRAME_NUMBER * _SAMPLE_NUM * 2 * _NBINS,), jnp.float32),
        mesh=mesh,
        compiler_params=pltpu.CompilerParams(needs_layout_passes=False),
        scratch_types=[
            pltpu.VMEM((2 * _CHUNK,), jnp.int32),
            pltpu.VMEM((2 * _CHUNK,), jnp.int32),
            pltpu.VMEM((2, 1, _CHUNK), jnp.float32),
            pltpu.VMEM((2, 1, _CHUNK), jnp.float32),
            pltpu.VMEM((16,), jnp.float32),
            pltpu.VMEM((_ACC,), jnp.float32),
            pltpu.SemaphoreType.DMA,
            pltpu.SemaphoreType.DMA,
        ],
    )(event_values, idx1, idx2, scale)


def kernel(event_values, event_indices, use_soft):
    scale = jnp.ones((16,), jnp.float32) * jnp.where(use_soft, 0.0, 1.0)
    idx1 = event_indices[0, 1]
    idx2 = event_indices[0, 2]
    raw = _frame_hist(event_values, idx1, idx2, scale)
    return raw.reshape(_FRAME_NUMBER, _SAMPLE_NUM, 2, _FRAME_SIZE, _FRAME_SIZE)
